# baseline (speedup 1.0000x reference)
import jax
import jax.numpy as jnp
from jax.experimental import pallas as pl

D = 768
NUM_EXPERTS = 64
TOP_K = 2
ROUTER_DIM = 16
TOKENS = 32768

BLOCK = 2048
HALF = TOKENS // 2


def _top2_rows(scores):
    st = scores.T
    eidx = jax.lax.broadcasted_iota(jnp.int32, st.shape, 0)
    m1 = jnp.max(st, axis=0, keepdims=True)
    i1 = jnp.min(jnp.where(st == m1, eidx, NUM_EXPERTS),
                 axis=0, keepdims=True)
    masked = jnp.where(eidx == i1, -jnp.inf, st)
    m2 = jnp.max(masked, axis=0, keepdims=True)
    i2 = jnp.min(jnp.where(masked == m2, eidx, NUM_EXPERTS),
                 axis=0, keepdims=True)
    idx = jnp.concatenate([i1, i2], axis=0)
    e = jnp.exp(m2 - m1)
    denom = 1.0 + e
    probs = jnp.concatenate([1.0 / denom, e / denom], axis=0)
    return idx, probs


def _router_block(xa_ref, xb_ref, wq_ref, keys_ref,
                  scores_ref, ia_ref, pa_ref, ib_ref, pb_ref):
    wq = wq_ref[...]
    keys = keys_ref[...]
    for h, (x, i_ref, p_ref) in enumerate(
            ((xa_ref[...], ia_ref, pa_ref), (xb_ref[...], ib_ref, pb_ref))):
        q = jax.lax.dot_general(
            x, wq, (((1,), (1,)), ((), ())),
            preferred_element_type=jnp.float32,
        )
        scores = jax.lax.dot_general(
            q, keys, (((1,), (1,)), ((), ())),
            preferred_element_type=jnp.float32,
        )
        scores_ref[h] = scores
        idx, probs = _top2_rows(scores)
        i_ref[...] = idx
        p_ref[...] = probs


@jax.jit
def kernel(x, W_query, keys):
    nb = HALF // BLOCK
    scores2, ia, pa, ib, pb = pl.pallas_call(
        _router_block,
        grid=(nb,),
        in_specs=[
            pl.BlockSpec((BLOCK, D), lambda i: (i, 0)),
            pl.BlockSpec((BLOCK, D), lambda i: (i + HALF // BLOCK, 0)),
            pl.BlockSpec((ROUTER_DIM, D), lambda i: (0, 0)),
            pl.BlockSpec((NUM_EXPERTS, ROUTER_DIM), lambda i: (0, 0)),
        ],
        out_specs=(
            pl.BlockSpec((2, BLOCK, NUM_EXPERTS), lambda i: (0, i, 0)),
            pl.BlockSpec((TOP_K, BLOCK), lambda i: (0, i)),
            pl.BlockSpec((TOP_K, BLOCK), lambda i: (0, i)),
            pl.BlockSpec((TOP_K, BLOCK), lambda i: (0, i)),
            pl.BlockSpec((TOP_K, BLOCK), lambda i: (0, i)),
        ),
        out_shape=(
            jax.ShapeDtypeStruct((2, HALF, NUM_EXPERTS), jnp.float32),
            jax.ShapeDtypeStruct((TOP_K, HALF), jnp.int32),
            jax.ShapeDtypeStruct((TOP_K, HALF), jnp.float32),
            jax.ShapeDtypeStruct((TOP_K, HALF), jnp.int32),
            jax.ShapeDtypeStruct((TOP_K, HALF), jnp.float32),
        ),
    )(x, x, W_query, keys)
    idx = jnp.concatenate([ia, ib], axis=1).T
    probs = jnp.concatenate([pa, pb], axis=1).T
    return idx, probs, scores2.reshape(TOKENS, NUM_EXPERTS)


# speedup vs baseline: 1.3109x; 1.3109x over previous
import jax
import jax.numpy as jnp
from jax.experimental import pallas as pl
from jax.experimental.pallas import tpu as pltpu

D = 768
NUM_EXPERTS = 64
TOP_K = 2
ROUTER_DIM = 16
TOKENS = 32768

BLOCK = 4096


def _router_block(x_ref, wq_ref, keys_ref, scores_ref, idx_ref, probs_ref):
    q = jax.lax.dot_general(
        x_ref[...], wq_ref[...], (((1,), (1,)), ((), ())),
        preferred_element_type=jnp.float32,
    )
    scores = jax.lax.dot_general(
        q, keys_ref[...], (((1,), (1,)), ((), ())),
        preferred_element_type=jnp.float32,
    )
    scores_ref[...] = scores

    st = scores.T
    eidx = jax.lax.broadcasted_iota(jnp.int32, st.shape, 0)
    m1 = jnp.max(st, axis=0, keepdims=True)
    i1 = jnp.min(jnp.where(st == m1, eidx, NUM_EXPERTS),
                 axis=0, keepdims=True)
    masked = jnp.where(eidx == i1, -jnp.inf, st)
    m2 = jnp.max(masked, axis=0, keepdims=True)
    i2 = jnp.min(jnp.where(masked == m2, eidx, NUM_EXPERTS),
                 axis=0, keepdims=True)

    idx_ref[...] = jnp.concatenate([i1, i2], axis=0)
    e = jnp.exp(m2 - m1)
    denom = 1.0 + e
    probs_ref[...] = jnp.concatenate([1.0 / denom, e / denom], axis=0)


@jax.jit
def kernel(x, W_query, keys):
    scores, idx2, probs2 = pl.pallas_call(
        _router_block,
        grid=(TOKENS // BLOCK,),
        in_specs=[
            pl.BlockSpec((BLOCK, D), lambda i: (i, 0)),
            pl.BlockSpec((ROUTER_DIM, D), lambda i: (0, 0)),
            pl.BlockSpec((NUM_EXPERTS, ROUTER_DIM), lambda i: (0, 0)),
        ],
        out_specs=(
            pl.BlockSpec((BLOCK, NUM_EXPERTS), lambda i: (i, 0)),
            pl.BlockSpec((TOP_K, BLOCK), lambda i: (0, i)),
            pl.BlockSpec((TOP_K, BLOCK), lambda i: (0, i)),
        ),
        out_shape=(
            jax.ShapeDtypeStruct((TOKENS, NUM_EXPERTS), jnp.float32),
            jax.ShapeDtypeStruct((TOP_K, TOKENS), jnp.int32),
            jax.ShapeDtypeStruct((TOP_K, TOKENS), jnp.float32),
        ),
        compiler_params=pltpu.CompilerParams(
            vmem_limit_bytes=100 * 1024 * 1024,
        ),
    )(x, W_query, keys)
    return idx2.T, probs2.T, scores
